# baseline (speedup 1.0000x reference)
import functools

import jax
import jax.numpy as jnp
from jax import lax
from jax.experimental import pallas as pl
from jax.experimental.pallas import tpu as pltpu
from jax.experimental.pallas import tpu_sc as plsc

B = 16384
D = 16
NC = 2
NS = 16
NW = NC * NS
BPW = B // NW

RATING_MIN = 1.0
RATING_MAX = 5.0


RB = 16


def _gather_body(user_hbm, item_hbm, umlp_hbm, imlp_hbm, umf_hbm, imf_hbm,
                 out_hbm, uidx, iidx, stage, rbufs, sems):
    wid = lax.axis_index("s") * NC + lax.axis_index("c")
    base = wid * BPW
    pltpu.sync_copy(user_hbm.at[pl.ds(base, BPW)], uidx.at[pl.ds(0, BPW)])
    pltpu.sync_copy(item_hbm.at[pl.ds(base, BPW)], iidx.at[pl.ds(0, BPW)])
    iota = lax.iota(jnp.int32, 16)

    for t, (table, idx_ref) in enumerate((
        (umlp_hbm, uidx),
        (imlp_hbm, iidx),
        (umf_hbm, uidx),
        (imf_hbm, iidx),
    )):
        def fire(e, k, table=table, idx_ref=idx_ref):
            i = idx_ref[pl.ds(e, 16)][0]
            col = pl.multiple_of(lax.shift_right_logical(i, 7) * 128, 128)
            pltpu.async_copy(table.at[:, pl.ds(col, 128)], rbufs[k], sems[k])

        def wait(k, table=table):
            pltpu.make_async_copy(
                table.at[:, pl.ds(0, 128)], rbufs[k], sems[k]).wait()

        def extract(e, k, idx_ref=idx_ref, t=t):
            i = idx_ref[pl.ds(e, 16)][0]
            lane = jnp.broadcast_to(jnp.bitwise_and(i, 127), (16,))
            vals = plsc.load_gather(rbufs[k], [iota, lane])
            plsc.store_scatter(
                stage, [iota + t * D, jnp.broadcast_to(e, (16,))], vals)

        for k in range(RB):
            fire(k, k)

        def group(g, _):
            for k in range(RB):
                e = g * RB + k
                wait(k)
                extract(e, k)
                fire(e + RB, k)
            return ()

        lax.fori_loop(0, BPW // RB - 1, group, (), unroll=False)
        for k in range(RB):
            e = BPW - RB + k
            wait(k)
            extract(e, k)

    pltpu.sync_copy(stage, out_hbm.at[:, pl.ds(base, BPW)])


def _gather4(user, item, t_umlp, t_imlp, t_umf, t_imf):
    mesh = plsc.VectorSubcoreMesh(core_axis_name="c", subcore_axis_name="s")
    f = functools.partial(
        pl.kernel,
        mesh=mesh,
        out_type=jax.ShapeDtypeStruct((4 * D, B), jnp.float32),
        compiler_params=pltpu.CompilerParams(
            use_tc_tiling_on_sc=True, needs_layout_passes=False),
        scratch_types=[
            pltpu.VMEM((BPW + 16,), jnp.int32),
            pltpu.VMEM((BPW + 16,), jnp.int32),
            pltpu.VMEM((4 * D, BPW), jnp.float32),
            [pltpu.VMEM((D, 128), jnp.float32) for _ in range(RB)],
            [pltpu.SemaphoreType.DMA for _ in range(RB)],
        ],
    )(_gather_body)
    return f(user, item, t_umlp.T, t_imlp.T, t_umf.T, t_imf.T)


BBLK = 16384


def _dense_body(x_ref, r_ref, w1_ref, w2_ref, w3_ref, loss_ref, tgt_ref):
    @pl.when(pl.program_id(0) == 0)
    def _init():
        loss_ref[...] = jnp.zeros((1, 1), dtype=jnp.float32)

    h = x_ref[pl.ds(0, 2 * D), :]
    h = jnp.tanh(jnp.dot(w1_ref[...], h, preferred_element_type=jnp.float32))
    h = jnp.tanh(jnp.dot(w2_ref[...], h, preferred_element_type=jnp.float32))
    mlp = jnp.tanh(jnp.dot(w3_ref[...], h, preferred_element_type=jnp.float32))
    u = x_ref[pl.ds(2 * D, D), :]
    v = x_ref[pl.ds(3 * D, D), :]
    u = u - jnp.mean(u, axis=0, keepdims=True)
    v = v - jnp.mean(v, axis=0, keepdims=True)
    un = jnp.maximum(jnp.sqrt(jnp.sum(u * u, axis=0, keepdims=True)), 1e-12)
    vn = jnp.maximum(jnp.sqrt(jnp.sum(v * v, axis=0, keepdims=True)), 1e-12)
    mf = jnp.sum(u * v, axis=0, keepdims=True) / (un * vn)
    nmf = 0.5 * mlp + 0.5 * mf
    r = (r_ref[...] - RATING_MIN) * (1.0 / (RATING_MAX - RATING_MIN))
    loss_ref[...] += jnp.full((1, 1), jnp.sum((nmf - r) ** 2) * (1.0 / B),
                              dtype=jnp.float32)
    tgt_ref[...] = nmf * (RATING_MAX - RATING_MIN) + RATING_MIN


def _dense(x, rating2, w1, w2, w3):
    return pl.pallas_call(
        _dense_body,
        grid=(B // BBLK,),
        in_specs=[
            pl.BlockSpec((4 * D, BBLK), lambda i: (0, i)),
            pl.BlockSpec((1, BBLK), lambda i: (0, i)),
            pl.BlockSpec((64, 32), lambda i: (0, 0)),
            pl.BlockSpec((32, 64), lambda i: (0, 0)),
            pl.BlockSpec((1, 32), lambda i: (0, 0)),
        ],
        out_specs=(
            pl.BlockSpec((1, 1), lambda i: (0, 0)),
            pl.BlockSpec((1, BBLK), lambda i: (0, i)),
        ),
        out_shape=(
            jax.ShapeDtypeStruct((1, 1), jnp.float32),
            jax.ShapeDtypeStruct((1, B), jnp.float32),
        ),
    )(x, rating2, w1, w2, w3)


def kernel(user, item, rating, user_weight_mlp, item_weight_mlp,
           user_weight_mf, item_weight_mf, W1, W2, W3):
    x = _gather4(user, item, user_weight_mlp, item_weight_mlp,
                 user_weight_mf, item_weight_mf)
    loss2, tgt2 = _dense(x, rating.reshape(1, B), W1, W2, W3)
    return loss2[0, 0], tgt2.reshape(B)


# speedup vs baseline: 1.1731x; 1.1731x over previous
import functools

import jax
import jax.numpy as jnp
from jax import lax
from jax.experimental import pallas as pl
from jax.experimental.pallas import tpu as pltpu
from jax.experimental.pallas import tpu_sc as plsc

B = 16384
D = 16
NC = 2
NS = 16
NW = NC * NS
BPW = B // NW

RATING_MIN = 1.0
RATING_MAX = 5.0


RB = 8


def _gather_body(user_hbm, item_hbm, umlp_hbm, imlp_hbm, umf_hbm, imf_hbm,
                 out_hbm, uidx, iidx, stage, rbufs, sems):
    wid = lax.axis_index("s") * NC + lax.axis_index("c")
    base = wid * BPW
    pltpu.sync_copy(user_hbm.at[pl.ds(base, BPW)], uidx.at[pl.ds(0, BPW)])
    pltpu.sync_copy(item_hbm.at[pl.ds(base, BPW)], iidx.at[pl.ds(0, BPW)])
    iota = lax.iota(jnp.int32, 16)

    tables = (
        (umlp_hbm, uidx),
        (imlp_hbm, iidx),
        (umf_hbm, uidx),
        (imf_hbm, iidx),
    )

    def fire(t, e, k):
        table, idx_ref = tables[t]
        i = idx_ref[pl.ds(e, 16)][0]
        col = pl.multiple_of(lax.shift_right_logical(i, 7) * 128, 128)
        pltpu.async_copy(table.at[:, pl.ds(col, 128)], rbufs[k], sems[k])

    def wait(k):
        pltpu.make_async_copy(
            tables[0][0].at[:, pl.ds(0, 128)], rbufs[k], sems[k]).wait()

    def extract(t, e, k):
        _, idx_ref = tables[t]
        i = idx_ref[pl.ds(e, 16)][0]
        lane = jnp.broadcast_to(jnp.bitwise_and(i, 127), (16,))
        vals = plsc.load_gather(rbufs[k], [iota, lane])
        plsc.store_scatter(
            stage, [iota + t * D, jnp.broadcast_to(e, (16,))], vals)

    for k in range(RB):
        fire(0, k, k)

    for t in range(4):
        def group(g, _, t=t):
            for k in range(RB):
                e = g * RB + k
                wait(k)
                extract(t, e, k)
                fire(t, e + RB, k)
            return ()

        lax.fori_loop(0, BPW // RB - 1, group, (), unroll=False)
        for k in range(RB):
            e = BPW - RB + k
            wait(k)
            extract(t, e, k)
            if t + 1 < 4:
                fire(t + 1, k, k)

    pltpu.sync_copy(stage, out_hbm.at[:, pl.ds(base, BPW)])


def _gather4(user, item, t_umlp, t_imlp, t_umf, t_imf):
    mesh = plsc.VectorSubcoreMesh(core_axis_name="c", subcore_axis_name="s")
    f = functools.partial(
        pl.kernel,
        mesh=mesh,
        out_type=jax.ShapeDtypeStruct((4 * D, B), jnp.float32),
        compiler_params=pltpu.CompilerParams(
            use_tc_tiling_on_sc=True, needs_layout_passes=False),
        scratch_types=[
            pltpu.VMEM((BPW + 16,), jnp.int32),
            pltpu.VMEM((BPW + 16,), jnp.int32),
            pltpu.VMEM((4 * D, BPW), jnp.float32),
            [pltpu.VMEM((D, 128), jnp.float32) for _ in range(RB)],
            [pltpu.SemaphoreType.DMA for _ in range(RB)],
        ],
    )(_gather_body)
    return f(user, item, t_umlp.T, t_imlp.T, t_umf.T, t_imf.T)


BBLK = 16384


def _dense_body(x_ref, r_ref, w1_ref, w2_ref, w3_ref, loss_ref, tgt_ref):
    @pl.when(pl.program_id(0) == 0)
    def _init():
        loss_ref[...] = jnp.zeros((1, 1), dtype=jnp.float32)

    h = x_ref[pl.ds(0, 2 * D), :]
    h = jnp.tanh(jnp.dot(w1_ref[...], h, preferred_element_type=jnp.float32))
    h = jnp.tanh(jnp.dot(w2_ref[...], h, preferred_element_type=jnp.float32))
    mlp = jnp.tanh(jnp.dot(w3_ref[...], h, preferred_element_type=jnp.float32))
    u = x_ref[pl.ds(2 * D, D), :]
    v = x_ref[pl.ds(3 * D, D), :]
    u = u - jnp.mean(u, axis=0, keepdims=True)
    v = v - jnp.mean(v, axis=0, keepdims=True)
    un = jnp.maximum(jnp.sqrt(jnp.sum(u * u, axis=0, keepdims=True)), 1e-12)
    vn = jnp.maximum(jnp.sqrt(jnp.sum(v * v, axis=0, keepdims=True)), 1e-12)
    mf = jnp.sum(u * v, axis=0, keepdims=True) / (un * vn)
    nmf = 0.5 * mlp + 0.5 * mf
    r = (r_ref[...] - RATING_MIN) * (1.0 / (RATING_MAX - RATING_MIN))
    loss_ref[...] += jnp.full((1, 1), jnp.sum((nmf - r) ** 2) * (1.0 / B),
                              dtype=jnp.float32)
    tgt_ref[...] = nmf * (RATING_MAX - RATING_MIN) + RATING_MIN


def _dense(x, rating2, w1, w2, w3):
    return pl.pallas_call(
        _dense_body,
        grid=(B // BBLK,),
        in_specs=[
            pl.BlockSpec((4 * D, BBLK), lambda i: (0, i)),
            pl.BlockSpec((1, BBLK), lambda i: (0, i)),
            pl.BlockSpec((64, 32), lambda i: (0, 0)),
            pl.BlockSpec((32, 64), lambda i: (0, 0)),
            pl.BlockSpec((1, 32), lambda i: (0, 0)),
        ],
        out_specs=(
            pl.BlockSpec((1, 1), lambda i: (0, 0)),
            pl.BlockSpec((1, BBLK), lambda i: (0, i)),
        ),
        out_shape=(
            jax.ShapeDtypeStruct((1, 1), jnp.float32),
            jax.ShapeDtypeStruct((1, B), jnp.float32),
        ),
    )(x, rating2, w1, w2, w3)


def kernel(user, item, rating, user_weight_mlp, item_weight_mlp,
           user_weight_mf, item_weight_mf, W1, W2, W3):
    x = _gather4(user, item, user_weight_mlp, item_weight_mlp,
                 user_weight_mf, item_weight_mf)
    loss2, tgt2 = _dense(x, rating.reshape(1, B), W1, W2, W3)
    return loss2[0, 0], tgt2.reshape(B)
